# baseline (speedup 1.0000x reference)
import functools

import jax
import jax.numpy as jnp
from jax import lax
from jax.experimental import pallas as pl
from jax.experimental.pallas import tpu as pltpu
from jax.experimental.pallas import tpu_sc as plsc

N = 150000
IN_CH = 16
OUT_CH = 16
BB = 2
DD = 41
HH = 400
WW = 352
NGRID = BB * DD * HH * WW
K27 = 27

NC = 2
NS = 16
NWK = NC * NS
NPAD = 150016
CHUNK = NPAD // NWK
NGROUPS = CHUNK // 16

_OFFSETS = [(dz, dy, dx) for dz in (-1, 0, 1) for dy in (-1, 0, 1) for dx in (-1, 0, 1)]
_CENTER = 13

_mesh = plsc.VectorSubcoreMesh(
    core_axis_name="c", subcore_axis_name="s", num_cores=NC, num_subcores=NS)


def _worker_base():
  wid = lax.axis_index("s") * NC + lax.axis_index("c")
  return wid * CHUNK


def _sc_scatter_body(bcol, dcol, hcol, wcol, grid, cb, cd, ch, cw, lin_v, val_v, sem):
  base = _worker_base()
  pltpu.sync_copy(bcol.at[pl.ds(base, CHUNK)], cb)
  pltpu.sync_copy(dcol.at[pl.ds(base, CHUNK)], cd)
  pltpu.sync_copy(hcol.at[pl.ds(base, CHUNK)], ch)
  pltpu.sync_copy(wcol.at[pl.ds(base, CHUNK)], cw)

  def body(g, carry):
    sl = pl.ds(g * 16, 16)
    lin = ((cb[sl] * DD + cd[sl]) * HH + ch[sl]) * WW + cw[sl]
    gi = base + g * 16 + lax.iota(jnp.int32, 16)
    lin_v[sl] = lin
    val_v[sl] = jnp.where(gi < N, gi, 0)
    return carry

  lax.fori_loop(0, NGROUPS, body, 0)
  pltpu.async_copy(val_v, grid.at[lin_v], sem).wait()


def _sc_gather_body(bcol, dcol, hcol, wcol, h27, grid, out,
                    cb, cd, ch, cw, lin_v, nlin_v, cand_v, hidx_v, acc_v, sem):
  base = _worker_base()
  pltpu.sync_copy(bcol.at[pl.ds(base, CHUNK)], cb)
  pltpu.sync_copy(dcol.at[pl.ds(base, CHUNK)], cd)
  pltpu.sync_copy(hcol.at[pl.ds(base, CHUNK)], ch)
  pltpu.sync_copy(wcol.at[pl.ds(base, CHUNK)], cw)

  def body0(g, carry):
    sl = pl.ds(g * 16, 16)
    lin_v[sl] = ((cb[sl] * DD + cd[sl]) * HH + ch[sl]) * WW + cw[sl]
    gi = base + g * 16 + lax.iota(jnp.int32, 16)
    hidx_v[sl] = gi * K27 + _CENTER
    return carry

  lax.fori_loop(0, NGROUPS, body0, 0)
  pltpu.async_copy(h27.at[hidx_v], acc_v, sem).wait()

  def mk_body1(dz, dy, dx, ck):
    def body1(g, carry):
      sl = pl.ds(g * 16, 16)
      nd = cd[sl] + dz
      nh = ch[sl] + dy
      nw = cw[sl] + dx
      ok = (nd >= 0) & (nd < DD) & (nh >= 0) & (nh < HH) & (nw >= 0) & (nw < WW)
      nlin_v[sl] = jnp.where(ok, lin_v[sl] + ck, NGRID)
      return carry
    return body1

  def mk_body2(k):
    def body2(g, carry):
      sl = pl.ds(g * 16, 16)
      c = cand_v[sl]
      hidx_v[sl] = jnp.where(c >= 0, c, N) * K27 + k
      return carry
    return body2

  for k, (dz, dy, dx) in enumerate(_OFFSETS):
    if k == _CENTER:
      continue
    ck = (dz * HH + dy) * WW + dx
    with jax.named_scope(f"nlin{k}"):
      lax.fori_loop(0, NGROUPS, mk_body1(dz, dy, dx, ck), 0)
    with jax.named_scope(f"grid{k}"):
      pltpu.async_copy(grid.at[nlin_v], cand_v, sem).wait()
    with jax.named_scope(f"sel{k}"):
      lax.fori_loop(0, NGROUPS, mk_body2(k), 0)
    with jax.named_scope(f"hadd{k}"):
      pltpu.async_copy(h27.at[hidx_v], acc_v, sem, add=True).wait()

  pltpu.sync_copy(acc_v, out.at[pl.ds(base, CHUNK)])


_sc_params = pltpu.CompilerParams(use_tc_tiling_on_sc=False)

_sc_scatter = functools.partial(
    pl.kernel,
    out_type=(),
    mesh=_mesh,
    compiler_params=_sc_params,
    scratch_types=[
        pltpu.VMEM((CHUNK,), jnp.int32),
        pltpu.VMEM((CHUNK,), jnp.int32),
        pltpu.VMEM((CHUNK,), jnp.int32),
        pltpu.VMEM((CHUNK,), jnp.int32),
        pltpu.VMEM((CHUNK,), jnp.int32),
        pltpu.VMEM((CHUNK,), jnp.int32),
        pltpu.SemaphoreType.DMA,
    ],
)(_sc_scatter_body)

_sc_gather = functools.partial(
    pl.kernel,
    out_type=jax.ShapeDtypeStruct((NPAD, OUT_CH), jnp.float32),
    mesh=_mesh,
    compiler_params=_sc_params,
    scratch_types=[
        pltpu.VMEM((CHUNK,), jnp.int32),
        pltpu.VMEM((CHUNK,), jnp.int32),
        pltpu.VMEM((CHUNK,), jnp.int32),
        pltpu.VMEM((CHUNK,), jnp.int32),
        pltpu.VMEM((CHUNK,), jnp.int32),
        pltpu.VMEM((CHUNK,), jnp.int32),
        pltpu.VMEM((CHUNK,), jnp.int32),
        pltpu.VMEM((CHUNK,), jnp.int32),
        pltpu.VMEM((CHUNK, OUT_CH), jnp.float32),
        pltpu.SemaphoreType.DMA,
    ],
)(_sc_gather_body)


_MM_BLK = 2344


def _mm_body(f_ref, w_ref, o_ref):
  o_ref[...] = jnp.dot(f_ref[...], w_ref[...], preferred_element_type=jnp.float32)


_mm = pl.pallas_call(
    _mm_body,
    grid=(NPAD // _MM_BLK,),
    in_specs=[
        pl.BlockSpec((_MM_BLK, IN_CH), lambda i: (i, 0)),
        pl.BlockSpec((IN_CH, K27 * OUT_CH), lambda i: (0, 0)),
    ],
    out_specs=pl.BlockSpec((_MM_BLK, K27 * OUT_CH), lambda i: (i, 0)),
    out_shape=jax.ShapeDtypeStruct((NPAD, K27 * OUT_CH), jnp.float32),
)


NROWS = NPAD * OUT_CH // 128
NROWS_VALID = N * OUT_CH // 128


def _bn_body(x_ref, g_ref, b_ref, y_ref):
  x = x_ref[...]
  rid = lax.broadcasted_iota(jnp.int32, (NROWS, 128), 0)
  m = (rid < NROWS_VALID).astype(jnp.float32)
  xm = x * m
  s = jnp.sum(xm, axis=0, keepdims=True)
  q = jnp.sum(xm * x, axis=0, keepdims=True)
  s_fold = s
  q_fold = q
  for j in range(1, 8):
    s_fold = s_fold + jnp.roll(s, 16 * j, axis=1)
    q_fold = q_fold + jnp.roll(q, 16 * j, axis=1)
  mean = s_fold * (1.0 / N)
  var = q_fold * (1.0 / N) - mean * mean
  inv = lax.rsqrt(var + 1e-5)
  scale = inv * g_ref[...]
  shift = b_ref[...] - mean * scale
  y_ref[...] = jnp.maximum(x * scale + shift, 0.0)


_bn = pl.pallas_call(
    _bn_body,
    out_shape=jax.ShapeDtypeStruct((NROWS, 128), jnp.float32),
)


def kernel(features, indices, W, gamma, beta):
  fpad = jnp.concatenate(
      [features, jnp.zeros((NPAD - N, IN_CH), jnp.float32)], axis=0)
  ipad = jnp.concatenate(
      [indices, jnp.broadcast_to(indices[0:1], (NPAD - N, 4))], axis=0)
  bcol = ipad[:, 0]
  dcol = ipad[:, 1]
  hcol = ipad[:, 2]
  wcol = ipad[:, 3]
  wcat = jnp.transpose(W, (1, 0, 2)).reshape(IN_CH, K27 * OUT_CH)

  h = _mm(fpad, wcat)
  h27 = h.reshape(NPAD * K27, OUT_CH)

  grid_ref = jax.new_ref(jnp.full((NGRID + 1,), -1, jnp.int32))
  _sc_scatter(bcol, dcol, hcol, wcol, grid_ref)
  conv = _sc_gather(bcol, dcol, hcol, wcol, h27, grid_ref)

  y = _bn(conv.reshape(NROWS, 128), jnp.tile(gamma, 8).reshape(1, 128),
          jnp.tile(beta, 8).reshape(1, 128))
  return y.reshape(NPAD, OUT_CH)[:N]


# speedup vs baseline: 9.8247x; 9.8247x over previous
import functools

import jax
import jax.numpy as jnp
from jax import lax
from jax.experimental import pallas as pl
from jax.experimental.pallas import tpu as pltpu
from jax.experimental.pallas import tpu_sc as plsc

N = 150000
IN_CH = 16
OUT_CH = 16
BB = 2
DD = 41
HH = 400
WW = 352
NGRID = BB * DD * HH * WW
K26 = 26

NC = 2
NS = 16
NWK = NC * NS
NPAD = 153600
CHUNK = NPAD // NWK
HCH = CHUNK // 2
NG = HCH // 16
QUARTER = NPAD // NS // 4
CAP = 2432
ADUMP = HCH + 16

BMW = 360832
BMZ = BMW // NS
SAFE_W = NGRID // 32

_OFFSETS26 = [(dz, dy, dx)
              for dz in (-1, 0, 1) for dy in (-1, 0, 1) for dx in (-1, 0, 1)
              if not (dz == 0 and dy == 0 and dx == 0)]

_mesh = plsc.VectorSubcoreMesh(
    core_axis_name="c", subcore_axis_name="s", num_cores=NC, num_subcores=NS)
_sc_params = pltpu.CompilerParams(
    use_tc_tiling_on_sc=False, needs_layout_passes=False)


def _iota16():
  return lax.iota(jnp.int32, 16)


def _sc_scatter_body(bcol, dcol, hcol, wcol, grid, cb, cd, ch, cw, lin_v, val_v, sem):
  wid = lax.axis_index("s") * NC + lax.axis_index("c")

  for h in range(2):
    base = wid * CHUNK + h * HCH
    pltpu.sync_copy(bcol.at[pl.ds(base, HCH)], cb)
    pltpu.sync_copy(dcol.at[pl.ds(base, HCH)], cd)
    pltpu.sync_copy(hcol.at[pl.ds(base, HCH)], ch)
    pltpu.sync_copy(wcol.at[pl.ds(base, HCH)], cw)

    def body(g, c2, base=base):
      sl = pl.ds(g * 16, 16)
      lin = ((cb[sl] * DD + cd[sl]) * HH + ch[sl]) * WW + cw[sl]
      gi = base + g * 16 + _iota16()
      lin_v[sl] = lin
      val_v[sl] = jnp.where(gi < N, gi, 0)
      return c2

    lax.fori_loop(0, NG, body, 0)
    pltpu.async_copy(val_v, grid.at[lin_v], sem).wait()


def _sc_gather_body(bcol, dcol, hcol, wcol, h26, hc, grid, out,
                    cb, cd, ch, cw, lin_v, nlin_v, wq_v, pword_v,
                    hidxc_v, dposc_v, cc_v, hh_v, bidx_v, gbuf_v, acc_v, bm, sem):
  sid = lax.axis_index("s")
  wid = sid * NC + lax.axis_index("c")

  def zbody(g, carry):
    pword_v[pl.ds(g * 16, 16)] = jnp.zeros((16,), jnp.int32)
    return carry
  lax.fori_loop(0, NG, zbody, 0)
  zoff = sid * BMZ
  for c in range(9):
    pltpu.sync_copy(pword_v.at[pl.ds(0, HCH)], bm.at[pl.ds(zoff + c * HCH, HCH)])
  pltpu.sync_copy(pword_v.at[pl.ds(0, BMZ - 9 * HCH)],
                  bm.at[pl.ds(zoff + 9 * HCH, BMZ - 9 * HCH)])
  plsc.subcore_barrier()

  def bpass(p, carry):
    hoff = sid * (4 * QUARTER) + p * QUARTER
    pltpu.sync_copy(bcol.at[pl.ds(hoff, HCH)], cb)
    pltpu.sync_copy(dcol.at[pl.ds(hoff, HCH)], cd)
    pltpu.sync_copy(hcol.at[pl.ds(hoff, HCH)], ch)
    pltpu.sync_copy(wcol.at[pl.ds(hoff, HCH)], cw)

    def bbody(g, c2):
      sl = pl.ds(g * 16, 16)
      lin = ((cb[sl] * DD + cd[sl]) * HH + ch[sl]) * WW + cw[sl]
      gi = hoff + g * 16 + _iota16()
      live = gi < N
      nlin_v[sl] = jnp.where(live, lin >> 5, SAFE_W)
      pword_v[sl] = jnp.where(live, 1 << (lin & 31), 0)
      return c2

    lax.fori_loop(0, NG, bbody, 0)
    pltpu.sync_copy(pword_v, bm.at[nlin_v], add=True)
    return carry

  lax.fori_loop(0, 4, bpass, 0)
  plsc.subcore_barrier()

  def prefill(g, carry):
    sl = pl.ds(g * 16, 16)
    hidxc_v[sl] = jnp.full((16,), NGRID, jnp.int32)
    dposc_v[sl] = HCH + _iota16()
    return carry

  def compact(g, cnt_vec):
    sl = pl.ds(g * 16, 16)
    w16 = pword_v[sl]
    nl16 = nlin_v[sl]
    hit = (w16 & (1 << (nl16 & 31))) != 0
    hi = hit.astype(jnp.int32)
    pos = cnt_vec - 1 + plsc.cumsum(hi)
    plsc.store_scatter(hidxc_v, [pos], nl16, mask=hit)
    plsc.store_scatter(dposc_v, [pos], g * 16 + _iota16(), mask=hit)
    return cnt_vec + plsc.all_reduce_population_count(hit)

  def half(hsel, carry):
    base = wid * CHUNK + hsel * HCH
    pltpu.sync_copy(bcol.at[pl.ds(base, HCH)], cb)
    pltpu.sync_copy(dcol.at[pl.ds(base, HCH)], cd)
    pltpu.sync_copy(hcol.at[pl.ds(base, HCH)], ch)
    pltpu.sync_copy(wcol.at[pl.ds(base, HCH)], cw)

    def lbody(g, c2):
      sl = pl.ds(g * 16, 16)
      lin_v[sl] = ((cb[sl] * DD + cd[sl]) * HH + ch[sl]) * WW + cw[sl]
      return c2
    lax.fori_loop(0, NG, lbody, 0)

    pltpu.sync_copy(hc.at[pl.ds(base, HCH)], acc_v.at[pl.ds(0, HCH)])

    def offs(j, c2):
      k = j + (j >= 13).astype(jnp.int32)
      dz = k // 9 - 1
      dy = (k // 3) % 3 - 1
      dx = k % 3 - 1
      ck = (dz * HH + dy) * WW + dx

      def nlin_f(g, c3):
        sl = pl.ds(g * 16, 16)
        nd = cd[sl] + dz
        nh = ch[sl] + dy
        nw = cw[sl] + dx
        ok = (nd >= 0) & (nd < DD) & (nh >= 0) & (nh < HH) & (nw >= 0) & (nw < WW)
        nl = jnp.where(ok, lin_v[sl] + ck, NGRID)
        nlin_v[sl] = nl
        wq_v[sl] = nl >> 5
        return c3

      lax.fori_loop(0, NG, nlin_f, 0)
      pltpu.async_copy(bm.at[wq_v], pword_v, sem).wait()
      lax.fori_loop(0, CAP // 16, prefill, 0)
      cnt_vec = lax.fori_loop(0, NG, compact, jnp.zeros((16,), jnp.int32))
      cnt = jnp.max(cnt_vec)
      nb = (cnt + 127) // 128

      def batch(bj, c3):
        def stage(q, c4):
          sl = pl.ds(q * 16, 16)
          bidx_v[sl] = hidxc_v[pl.ds(bj * 128 + q * 16, 16)]
          return c4
        lax.fori_loop(0, 8, stage, 0)
        pltpu.async_copy(grid.at[bidx_v], cc_v, sem).wait()

        def rows(q, c4):
          sl = pl.ds(q * 16, 16)
          c16 = cc_v[sl]
          hh_v[sl] = jnp.where(c16 >= 0, c16, N) * K26 + j
          return c4
        lax.fori_loop(0, 8, rows, 0)
        pltpu.async_copy(h26.at[hh_v], gbuf_v, sem).wait()

        def sadd(q, c4):
          p16 = dposc_v[pl.ds(bj * 128 + q * 16, 16)]
          src = q * 16 + _iota16()

          def chan(cix, c5):
            cvec = jnp.full((16,), cix, jnp.int32)
            vals = plsc.load_gather(gbuf_v, [src, cvec])
            plsc.addupdate_scatter(acc_v, [p16, cvec], vals)
            return c5
          lax.fori_loop(0, OUT_CH, chan, 0)
          return c4
        lax.fori_loop(0, 8, sadd, 0)
        return c3

      lax.fori_loop(0, nb, batch, 0)
      return c2

    lax.fori_loop(0, K26, offs, 0)
    pltpu.sync_copy(acc_v.at[pl.ds(0, HCH)], out.at[pl.ds(base, HCH)])
    return carry

  lax.fori_loop(0, 2, half, 0)


_sc_scatter = functools.partial(
    pl.kernel,
    out_type=(),
    mesh=_mesh,
    compiler_params=_sc_params,
    scratch_types=[
        pltpu.VMEM((HCH,), jnp.int32),
        pltpu.VMEM((HCH,), jnp.int32),
        pltpu.VMEM((HCH,), jnp.int32),
        pltpu.VMEM((HCH,), jnp.int32),
        pltpu.VMEM((HCH,), jnp.int32),
        pltpu.VMEM((HCH,), jnp.int32),
        pltpu.SemaphoreType.DMA,
    ],
)(_sc_scatter_body)

_sc_gather = functools.partial(
    pl.kernel,
    out_type=jax.ShapeDtypeStruct((NPAD, OUT_CH), jnp.float32),
    mesh=_mesh,
    compiler_params=_sc_params,
    scratch_types=[
        pltpu.VMEM((HCH,), jnp.int32),
        pltpu.VMEM((HCH,), jnp.int32),
        pltpu.VMEM((HCH,), jnp.int32),
        pltpu.VMEM((HCH,), jnp.int32),
        pltpu.VMEM((HCH,), jnp.int32),
        pltpu.VMEM((HCH,), jnp.int32),
        pltpu.VMEM((HCH,), jnp.int32),
        pltpu.VMEM((HCH,), jnp.int32),
        pltpu.VMEM((CAP,), jnp.int32),
        pltpu.VMEM((CAP,), jnp.int32),
        pltpu.VMEM((128,), jnp.int32),
        pltpu.VMEM((128,), jnp.int32),
        pltpu.VMEM((128,), jnp.int32),
        pltpu.VMEM((128, OUT_CH), jnp.float32),
        pltpu.VMEM((ADUMP, OUT_CH), jnp.float32),
        pltpu.VMEM_SHARED((BMW,), jnp.int32),
        pltpu.SemaphoreType.DMA,
    ],
)(_sc_gather_body)


_MM_BLK = 2400


def _mm_body(f_ref, w26_ref, wc_ref, o26_ref, oc_ref):
  f = f_ref[...]
  o26_ref[...] = jnp.dot(f, w26_ref[...], preferred_element_type=jnp.float32)
  oc_ref[...] = jnp.dot(f, wc_ref[...], preferred_element_type=jnp.float32)


_mm = pl.pallas_call(
    _mm_body,
    grid=(NPAD // _MM_BLK,),
    in_specs=[
        pl.BlockSpec((_MM_BLK, IN_CH), lambda i: (i, 0)),
        pl.BlockSpec((IN_CH, K26 * OUT_CH), lambda i: (0, 0)),
        pl.BlockSpec((IN_CH, OUT_CH), lambda i: (0, 0)),
    ],
    out_specs=[
        pl.BlockSpec((_MM_BLK, K26 * OUT_CH), lambda i: (i, 0)),
        pl.BlockSpec((_MM_BLK, OUT_CH), lambda i: (i, 0)),
    ],
    out_shape=[
        jax.ShapeDtypeStruct((NPAD, K26 * OUT_CH), jnp.float32),
        jax.ShapeDtypeStruct((NPAD, OUT_CH), jnp.float32),
    ],
)

NROWS = NPAD * OUT_CH // 128
NROWS_VALID = N * OUT_CH // 128


def _bn_body(x_ref, g_ref, b_ref, y_ref):
  x = x_ref[...]
  rid = lax.broadcasted_iota(jnp.int32, (NROWS, 128), 0)
  m = (rid < NROWS_VALID).astype(jnp.float32)
  xm = x * m
  s = jnp.sum(xm, axis=0, keepdims=True)
  q = jnp.sum(xm * x, axis=0, keepdims=True)
  s_fold = s
  q_fold = q
  for j in range(1, 8):
    s_fold = s_fold + jnp.roll(s, 16 * j, axis=1)
    q_fold = q_fold + jnp.roll(q, 16 * j, axis=1)
  mean = s_fold * (1.0 / N)
  var = q_fold * (1.0 / N) - mean * mean
  inv = lax.rsqrt(var + 1e-5)
  scale = inv * g_ref[...]
  shift = b_ref[...] - mean * scale
  y_ref[...] = jnp.maximum(x * scale + shift, 0.0)


_bn = pl.pallas_call(
    _bn_body,
    out_shape=jax.ShapeDtypeStruct((NROWS, 128), jnp.float32),
)


def kernel(features, indices, W, gamma, beta):
  fpad = jnp.concatenate(
      [features, jnp.zeros((NPAD - N, IN_CH), jnp.float32)], axis=0)
  ipad = jnp.concatenate(
      [indices, jnp.broadcast_to(indices[0:1], (NPAD - N, 4))], axis=0)
  bcol = ipad[:, 0]
  dcol = ipad[:, 1]
  hcol = ipad[:, 2]
  wcol = ipad[:, 3]
  w26 = jnp.concatenate([W[:13], W[14:]], axis=0)
  wcat26 = jnp.transpose(w26, (1, 0, 2)).reshape(IN_CH, K26 * OUT_CH)
  wc = W[13]

  h26, hcen = _mm(fpad, wcat26, wc)
  h26r = h26.reshape(NPAD * K26, OUT_CH)

  grid_ref = jax.new_ref(jnp.full((NGRID + 1,), -1, jnp.int32))
  _sc_scatter(bcol, dcol, hcol, wcol, grid_ref)
  conv = _sc_gather(bcol, dcol, hcol, wcol, h26r, hcen, grid_ref)

  y = _bn(conv.reshape(NROWS, 128), jnp.tile(gamma, 8).reshape(1, 128),
          jnp.tile(beta, 8).reshape(1, 128))
  return y.reshape(NPAD, OUT_CH)[:N]
